# baseline (speedup 1.0000x reference)
import functools

import jax
import jax.numpy as jnp
from jax import lax
from jax.experimental import pallas as pl
from jax.experimental.pallas import tpu as pltpu
from jax.experimental.pallas import tpu_sc as plsc

N = 10000
E = 320000
D = 128
G = 64
C = 10

NPAD = 10240
RPT = NPAD // 16
NW = 32
STEP = 128
NSTEPS = 84
HSTEPS = NSTEPS // 2
EPAD = NW * NSTEPS * STEP

_mesh = plsc.VectorSubcoreMesh(core_axis_name="c", subcore_axis_name="s")


@functools.partial(
    pl.kernel,
    mesh=_mesh,
    out_type=jax.ShapeDtypeStruct((2, NPAD), jnp.float32),
    scratch_types=[
        pltpu.VMEM((NSTEPS, STEP), jnp.int32),
        pltpu.VMEM((STEP,), jnp.float32),
        pltpu.VMEM_SHARED((NPAD,), jnp.float32),
        pltpu.SemaphoreType.DMA,
    ],
)
def _deg(dst_hbm, zeros1_hbm, out_hbm, dst_v, ones_v, acc, sem):
    c = lax.axis_index("c")
    s = lax.axis_index("s")
    wid = s * 2 + c
    for i in range(STEP // 16):
        ones_v[pl.ds(i * 16, 16)] = jnp.ones((16,), jnp.float32)
    pltpu.sync_copy(zeros1_hbm.at[pl.ds(s * RPT, RPT)],
                    acc.at[pl.ds(s * RPT, RPT)])
    pltpu.sync_copy(dst_hbm.at[wid], dst_v)
    plsc.subcore_barrier()

    def body(j, carry):
        pltpu.sync_copy(ones_v, acc.at[dst_v.at[j]], add=True)
        return carry

    lax.fori_loop(0, NSTEPS, body, 0)
    plsc.subcore_barrier()
    pltpu.sync_copy(acc.at[pl.ds(s * RPT, RPT)],
                    out_hbm.at[c, pl.ds(s * RPT, RPT)])


def _make_agg(H):
    @functools.partial(
        pl.kernel,
        mesh=_mesh,
        out_type=jax.ShapeDtypeStruct((2, NPAD, H), jnp.float32),
        scratch_types=[
            pltpu.VMEM((NSTEPS, STEP), jnp.int32),
            pltpu.VMEM((NSTEPS, STEP), jnp.int32),
            pltpu.VMEM((STEP, H), jnp.float32),
            pltpu.VMEM_SHARED((NPAD, H), jnp.float32),
            pltpu.SemaphoreType.DMA,
        ],
    )
    def _agg(zs_hbm, src_hbm, dst_hbm, zeros_hbm, out_hbm,
             src_v, dst_v, rows0, acc, gsem0):
        c = lax.axis_index("c")
        s = lax.axis_index("s")
        wid = s * 2 + c
        pltpu.sync_copy(zeros_hbm.at[pl.ds(s * RPT, RPT)],
                        acc.at[pl.ds(s * RPT, RPT)])
        pltpu.sync_copy(src_hbm.at[wid], src_v)
        pltpu.sync_copy(dst_hbm.at[wid], dst_v)
        plsc.subcore_barrier()

        def body(j, carry):
            pltpu.async_copy(zs_hbm.at[src_v.at[j]], rows0, gsem0).wait()
            pltpu.sync_copy(rows0, acc.at[dst_v.at[j]], add=True)
            return carry

        lax.fori_loop(0, NSTEPS, body, 0)
        plsc.subcore_barrier()
        pltpu.sync_copy(acc.at[pl.ds(s * RPT, RPT)],
                        out_hbm.at[c, pl.ds(s * RPT, RPT)])

    return _agg


_agg128 = _make_agg(128)


def _stage0_body(d0_ref, d1_ref, x_ref, w_ref, dinv_ref, zs_ref):
    deg = d0_ref[...] + d1_ref[...]
    dinv = lax.rsqrt(deg)
    dinv_ref[...] = dinv
    zs_ref[...] = jnp.dot(x_ref[...] * dinv, w_ref[...],
                          preferred_element_type=jnp.float32)


def _stage0(d0, d1, x, w):
    return pl.pallas_call(
        _stage0_body,
        out_shape=[jax.ShapeDtypeStruct((N, 1), jnp.float32),
                   jax.ShapeDtypeStruct((N, D), jnp.float32)],
    )(d0, d1, x, w)


def _stage_mid_body(a0_ref, a1_ref, dinv_ref, b_ref, g_ref, be_ref, w_ref,
                    zs_ref):
    dinv = dinv_ref[...]
    pre = (a0_ref[...] + a1_ref[...]) * dinv + b_ref[...]
    m = jnp.mean(pre, axis=0, keepdims=True)
    cen = pre - m
    var = jnp.mean(cen * cen, axis=0, keepdims=True)
    h = g_ref[...] * cen * lax.rsqrt(var + 1e-5) + be_ref[...]
    h = jnp.maximum(h, 0.0)
    zs_ref[...] = jnp.dot(h * dinv, w_ref[...],
                          preferred_element_type=jnp.float32)


def _stage_mid(a0, a1, dinv, b, g, be, w):
    hin, hout = w.shape
    return pl.pallas_call(
        _stage_mid_body,
        out_shape=jax.ShapeDtypeStruct((N, hout), jnp.float32),
    )(a0, a1, dinv, b.reshape(1, hin), g.reshape(1, hin), be.reshape(1, hin), w)


def _stage_final_body(a0_ref, a1_ref, dinv_ref, b_ref, g_ref, be_ref,
                      batch_ref, wl_ref, bl_ref, out_ref):
    pre = (a0_ref[...] + a1_ref[...]) * dinv_ref[...] + b_ref[...]
    m = jnp.mean(pre, axis=0, keepdims=True)
    cen = pre - m
    var = jnp.mean(cen * cen, axis=0, keepdims=True)
    h = g_ref[...] * cen * lax.rsqrt(var + 1e-5) + be_ref[...]
    h = jnp.maximum(h, 0.0)
    gids = lax.broadcasted_iota(jnp.int32, (1, G), 1)
    onehot = (batch_ref[...] == gids).astype(jnp.float32)
    counts = jnp.sum(onehot, axis=0)
    sums = lax.dot_general(onehot, h, (((0,), (0,)), ((), ())),
                           preferred_element_type=jnp.float32)
    pooled = sums / jnp.maximum(counts, 1.0)[:, None]
    logits = jnp.dot(pooled, wl_ref[...],
                     preferred_element_type=jnp.float32) + bl_ref[...]
    mx = jnp.max(logits, axis=1, keepdims=True)
    e = jnp.exp(logits - mx)
    out_ref[...] = e / jnp.sum(e, axis=1, keepdims=True)


def _stage_final(a0, a1, dinv, b, g, be, batch, wl, bl):
    hw = b.shape[0]
    return pl.pallas_call(
        _stage_final_body,
        out_shape=jax.ShapeDtypeStruct((G, C), jnp.float32),
    )(a0, a1, dinv, b.reshape(1, hw), g.reshape(1, hw), be.reshape(1, hw),
      batch.reshape(N, 1), wl, bl.reshape(1, C))


def kernel(x, edge_index, batch, W1, b1, g1, be1, W2, b2, g2, be2,
           W4, b4, g4, be4, W3, b3, g3, be3, Wl, bl):
    loop = jnp.arange(N, dtype=jnp.int32)
    pad = EPAD - (E + N)
    src = jnp.concatenate(
        [edge_index[0], loop, jnp.zeros((pad,), jnp.int32)])
    dst = jnp.concatenate(
        [edge_index[1], loop, jnp.full((pad,), N, jnp.int32)])
    src4 = src.reshape(NW, NSTEPS, STEP)
    dst4 = dst.reshape(NW, NSTEPS, STEP)
    dst3 = dst4

    zeros1 = jnp.zeros((NPAD,), jnp.float32)
    zeros128 = jnp.zeros((NPAD, 128), jnp.float32)

    degp = _deg(dst3, zeros1)
    dinv, zs = _stage0(degp[0, :N, None], degp[1, :N, None], x, W1)

    a = _agg128(zs, src4, dst4, zeros128)
    zs = _stage_mid(a[0, :N], a[1, :N], dinv, b1, g1, be1, W2)
    a = _agg128(zs, src4, dst4, zeros128)
    zs = _stage_mid(a[0, :N], a[1, :N], dinv, b2, g2, be2, W4)
    a = _agg128(zs, src4, dst4, zeros128)
    zs = _stage_mid(a[0, :N], a[1, :N], dinv, b4, g4, be4,
                    jnp.pad(W3, ((0, 0), (0, 64))))
    a = _agg128(zs, src4, dst4, zeros128)
    return _stage_final(a[0, :N, :64], a[1, :N, :64], dinv, b3, g3, be3,
                        batch, Wl, bl)


# speedup vs baseline: 2.7299x; 2.7299x over previous
import functools

import jax
import jax.numpy as jnp
from jax import lax
from jax.experimental import pallas as pl
from jax.experimental.pallas import tpu as pltpu
from jax.experimental.pallas import tpu_sc as plsc

N = 10000
E = 320000
D = 128
G = 64
C = 10

NPAD = 10240
RPT = NPAD // 16
NW = 32
STEP = 128
NSTEPS = 81
EPAD = NW * NSTEPS * STEP

_mesh = plsc.VectorSubcoreMesh(core_axis_name="c", subcore_axis_name="s")


@functools.partial(
    pl.kernel,
    mesh=_mesh,
    out_type=jax.ShapeDtypeStruct((2, NPAD), jnp.float32),
    scratch_types=[
        pltpu.VMEM((NSTEPS, STEP), jnp.int32),
        pltpu.VMEM((STEP,), jnp.float32),
        pltpu.VMEM_SHARED((NPAD,), jnp.float32),
        pltpu.SemaphoreType.DMA,
    ],
)
def _deg(dst_hbm, zeros1_hbm, out_hbm, dst_v, ones_v, acc, sem):
    c = lax.axis_index("c")
    s = lax.axis_index("s")
    wid = s * 2 + c
    for i in range(STEP // 16):
        ones_v[pl.ds(i * 16, 16)] = jnp.ones((16,), jnp.float32)
    pltpu.sync_copy(zeros1_hbm.at[pl.ds(s * RPT, RPT)],
                    acc.at[pl.ds(s * RPT, RPT)])
    pltpu.sync_copy(dst_hbm.at[wid], dst_v)
    plsc.subcore_barrier()

    def body(j, carry):
        pltpu.sync_copy(ones_v, acc.at[dst_v.at[j]], add=True)
        return carry

    lax.fori_loop(0, NSTEPS, body, 0)
    plsc.subcore_barrier()
    pltpu.sync_copy(acc.at[pl.ds(s * RPT, RPT)],
                    out_hbm.at[c, pl.ds(s * RPT, RPT)])


def _make_agg(H):
    @functools.partial(
        pl.kernel,
        mesh=_mesh,
        out_type=jax.ShapeDtypeStruct((2, NPAD, H), jnp.float32),
        scratch_types=[
            pltpu.VMEM((NSTEPS, STEP), jnp.int32),
            pltpu.VMEM((NSTEPS, STEP), jnp.int32),
            pltpu.VMEM((STEP, H), jnp.float32),
            pltpu.VMEM_SHARED((NPAD, H), jnp.float32),
            pltpu.SemaphoreType.DMA,
        ],
    )
    def _agg(zs_hbm, src_hbm, dst_hbm, zeros_hbm, out_hbm,
             src_v, dst_v, rows0, acc, gsem0):
        c = lax.axis_index("c")
        s = lax.axis_index("s")
        wid = s * 2 + c
        pltpu.sync_copy(zeros_hbm.at[pl.ds(s * RPT, RPT)],
                        acc.at[pl.ds(s * RPT, RPT)])
        pltpu.sync_copy(src_hbm.at[wid], src_v)
        pltpu.sync_copy(dst_hbm.at[wid], dst_v)
        plsc.subcore_barrier()

        def body(j, carry):
            pltpu.async_copy(zs_hbm.at[src_v.at[j]], rows0, gsem0).wait()
            pltpu.sync_copy(rows0, acc.at[dst_v.at[j]], add=True)
            return carry

        lax.fori_loop(0, NSTEPS, body, 0)
        plsc.subcore_barrier()
        pltpu.sync_copy(acc.at[pl.ds(s * RPT, RPT)],
                        out_hbm.at[c, pl.ds(s * RPT, RPT)])

    return _agg


_agg128 = _make_agg(128)


def _stage0_body(d0_ref, d1_ref, x_ref, w_ref, dinv_ref, zs_ref):
    deg = d0_ref[...] + d1_ref[...]
    dinv = lax.rsqrt(deg)
    dinv_ref[...] = dinv
    zs_ref[...] = jnp.dot(x_ref[...] * dinv, w_ref[...],
                          preferred_element_type=jnp.float32)


def _stage0(d0, d1, x, w):
    return pl.pallas_call(
        _stage0_body,
        out_shape=[jax.ShapeDtypeStruct((N, 1), jnp.float32),
                   jax.ShapeDtypeStruct((N, D), jnp.float32)],
    )(d0, d1, x, w)


def _stage_mid_body(a0_ref, a1_ref, dinv_ref, b_ref, g_ref, be_ref, w_ref,
                    zs_ref):
    dinv = dinv_ref[...]
    pre = (a0_ref[...] + a1_ref[...]) * dinv + b_ref[...]
    m = jnp.mean(pre, axis=0, keepdims=True)
    cen = pre - m
    var = jnp.mean(cen * cen, axis=0, keepdims=True)
    h = g_ref[...] * cen * lax.rsqrt(var + 1e-5) + be_ref[...]
    h = jnp.maximum(h, 0.0)
    zs_ref[...] = jnp.dot(h * dinv, w_ref[...],
                          preferred_element_type=jnp.float32)


def _stage_mid(a0, a1, dinv, b, g, be, w):
    hin, hout = w.shape
    return pl.pallas_call(
        _stage_mid_body,
        out_shape=jax.ShapeDtypeStruct((N, hout), jnp.float32),
    )(a0, a1, dinv, b.reshape(1, hin), g.reshape(1, hin), be.reshape(1, hin), w)


def _stage_final_body(a0_ref, a1_ref, dinv_ref, b_ref, g_ref, be_ref,
                      batch_ref, wl_ref, bl_ref, out_ref):
    pre = (a0_ref[...] + a1_ref[...]) * dinv_ref[...] + b_ref[...]
    m = jnp.mean(pre, axis=0, keepdims=True)
    cen = pre - m
    var = jnp.mean(cen * cen, axis=0, keepdims=True)
    h = g_ref[...] * cen * lax.rsqrt(var + 1e-5) + be_ref[...]
    h = jnp.maximum(h, 0.0)
    gids = lax.broadcasted_iota(jnp.int32, (1, G), 1)
    onehot = (batch_ref[...] == gids).astype(jnp.float32)
    counts = jnp.sum(onehot, axis=0)
    sums = lax.dot_general(onehot, h, (((0,), (0,)), ((), ())),
                           preferred_element_type=jnp.float32)
    pooled = sums / jnp.maximum(counts, 1.0)[:, None]
    logits = jnp.dot(pooled, wl_ref[...],
                     preferred_element_type=jnp.float32) + bl_ref[...]
    mx = jnp.max(logits, axis=1, keepdims=True)
    e = jnp.exp(logits - mx)
    out_ref[...] = e / jnp.sum(e, axis=1, keepdims=True)


def _stage_final(a0, a1, dinv, b, g, be, batch, wl, bl):
    hw = b.shape[0]
    return pl.pallas_call(
        _stage_final_body,
        out_shape=jax.ShapeDtypeStruct((G, C), jnp.float32),
    )(a0, a1, dinv, b.reshape(1, hw), g.reshape(1, hw), be.reshape(1, hw),
      batch.reshape(N, 1), wl, bl.reshape(1, C))


def kernel(x, edge_index, batch, W1, b1, g1, be1, W2, b2, g2, be2,
           W4, b4, g4, be4, W3, b3, g3, be3, Wl, bl):
    loop = jnp.arange(N, dtype=jnp.int32)
    pad = EPAD - (E + N)
    src = jnp.concatenate(
        [edge_index[0], loop, jnp.zeros((pad,), jnp.int32)])
    dst = jnp.concatenate(
        [edge_index[1], loop,
         N + (jnp.arange(pad, dtype=jnp.int32) % (NPAD - N))])
    src4 = src.reshape(NW, NSTEPS, STEP)
    dst4 = dst.reshape(NW, NSTEPS, STEP)
    dst3 = dst4

    zeros1 = jnp.zeros((NPAD,), jnp.float32)
    zeros128 = jnp.zeros((NPAD, 128), jnp.float32)

    degp = _deg(dst3, zeros1)
    dinv, zs = _stage0(degp[0, :N, None], degp[1, :N, None], x, W1)

    a = _agg128(zs, src4, dst4, zeros128)
    zs = _stage_mid(a[0, :N], a[1, :N], dinv, b1, g1, be1, W2)
    a = _agg128(zs, src4, dst4, zeros128)
    zs = _stage_mid(a[0, :N], a[1, :N], dinv, b2, g2, be2, W4)
    a = _agg128(zs, src4, dst4, zeros128)
    zs = _stage_mid(a[0, :N], a[1, :N], dinv, b4, g4, be4,
                    jnp.pad(W3, ((0, 0), (0, 64))))
    a = _agg128(zs, src4, dst4, zeros128)
    return _stage_final(a[0, :N, :64], a[1, :N, :64], dinv, b3, g3, be3,
                        batch, Wl, bl)


# speedup vs baseline: 2.7839x; 1.0198x over previous
import functools

import jax
import jax.numpy as jnp
from jax import lax
from jax.experimental import pallas as pl
from jax.experimental.pallas import tpu as pltpu
from jax.experimental.pallas import tpu_sc as plsc

N = 10000
E = 320000
D = 128
G = 64
C = 10

NPAD = 10240
RPT = NPAD // 16
NW = 32
STEP = 128
NSTEPS = 81
EPAD = NW * NSTEPS * STEP

_mesh = plsc.VectorSubcoreMesh(core_axis_name="c", subcore_axis_name="s")


@functools.partial(
    pl.kernel,
    mesh=_mesh,
    out_type=jax.ShapeDtypeStruct((2, NPAD), jnp.float32),
    scratch_types=[
        pltpu.VMEM((NSTEPS, STEP), jnp.int32),
        pltpu.VMEM((STEP,), jnp.float32),
        pltpu.VMEM_SHARED((NPAD,), jnp.float32),
        pltpu.SemaphoreType.DMA,
    ],
)
def _deg(dst_hbm, zeros1_hbm, out_hbm, dst_v, ones_v, acc, sem):
    c = lax.axis_index("c")
    s = lax.axis_index("s")
    wid = s * 2 + c
    for i in range(STEP // 16):
        ones_v[pl.ds(i * 16, 16)] = jnp.ones((16,), jnp.float32)
    pltpu.sync_copy(zeros1_hbm.at[pl.ds(s * RPT, RPT)],
                    acc.at[pl.ds(s * RPT, RPT)])
    pltpu.sync_copy(dst_hbm.at[wid], dst_v)
    plsc.subcore_barrier()

    def body(j, carry):
        pltpu.sync_copy(ones_v, acc.at[dst_v.at[j]], add=True)
        return carry

    lax.fori_loop(0, NSTEPS, body, 0)
    plsc.subcore_barrier()
    pltpu.sync_copy(acc.at[pl.ds(s * RPT, RPT)],
                    out_hbm.at[c, pl.ds(s * RPT, RPT)])


def _make_agg(H):
    @functools.partial(
        pl.kernel,
        mesh=_mesh,
        out_type=jax.ShapeDtypeStruct((2, NPAD, H), jnp.float32),
        scratch_types=[
            pltpu.VMEM((NSTEPS, STEP), jnp.int32),
            pltpu.VMEM((NSTEPS, STEP), jnp.int32),
            pltpu.VMEM((STEP, H), jnp.float32),
            pltpu.VMEM_SHARED((NPAD, H), jnp.float32),
            pltpu.SemaphoreType.DMA,
        ],
    )
    def _agg(zs_hbm, src_hbm, dst_hbm, zeros_hbm, out_hbm,
             src_v, dst_v, rows0, acc, gsem0):
        c = lax.axis_index("c")
        s = lax.axis_index("s")
        wid = s * 2 + c
        pltpu.sync_copy(zeros_hbm.at[pl.ds(s * RPT, RPT)],
                        acc.at[pl.ds(s * RPT, RPT)])
        pltpu.sync_copy(src_hbm.at[wid], src_v)
        pltpu.sync_copy(dst_hbm.at[wid], dst_v)
        plsc.subcore_barrier()

        def body(j, carry):
            pltpu.async_copy(zs_hbm.at[src_v.at[j]], rows0, gsem0).wait()
            pltpu.sync_copy(rows0, acc.at[dst_v.at[j]], add=True)
            return carry

        lax.fori_loop(0, NSTEPS, body, 0)
        plsc.subcore_barrier()
        pltpu.sync_copy(acc.at[pl.ds(s * RPT, RPT)],
                        out_hbm.at[c, pl.ds(s * RPT, RPT)])

    return _agg


_agg128 = _make_agg(128)


def _stage0_body(d0_ref, d1_ref, x_ref, w_ref, dinv_ref, zs_ref):
    deg = d0_ref[...] + d1_ref[...]
    dinv = lax.rsqrt(deg)
    dinv_ref[...] = dinv
    zs_ref[...] = jnp.dot(x_ref[...] * dinv, w_ref[...],
                          preferred_element_type=jnp.float32)


def _stage0(d0, d1, x, w):
    return pl.pallas_call(
        _stage0_body,
        out_shape=[jax.ShapeDtypeStruct((N, 1), jnp.float32),
                   jax.ShapeDtypeStruct((N, D), jnp.float32)],
    )(d0, d1, x, w)


def _stage_mid_body(a0_ref, a1_ref, dinv_ref, b_ref, g_ref, be_ref, w_ref,
                    zs_ref):
    dinv = dinv_ref[...]
    pre = (a0_ref[:N] + a1_ref[:N]) * dinv + b_ref[...]
    m = jnp.mean(pre, axis=0, keepdims=True)
    cen = pre - m
    var = jnp.mean(cen * cen, axis=0, keepdims=True)
    h = g_ref[...] * cen * lax.rsqrt(var + 1e-5) + be_ref[...]
    h = jnp.maximum(h, 0.0)
    zs_ref[...] = jnp.dot(h * dinv, w_ref[...],
                          preferred_element_type=jnp.float32)


def _stage_mid(a0, a1, dinv, b, g, be, w):
    hin, hout = w.shape
    return pl.pallas_call(
        _stage_mid_body,
        out_shape=jax.ShapeDtypeStruct((N, hout), jnp.float32),
    )(a0, a1, dinv, b.reshape(1, hin), g.reshape(1, hin), be.reshape(1, hin), w)


def _stage_final_body(a0_ref, a1_ref, dinv_ref, b_ref, g_ref, be_ref,
                      batch_ref, wl_ref, bl_ref, out_ref):
    pre = (a0_ref[:N, :64] + a1_ref[:N, :64]) * dinv_ref[...] + b_ref[...]
    m = jnp.mean(pre, axis=0, keepdims=True)
    cen = pre - m
    var = jnp.mean(cen * cen, axis=0, keepdims=True)
    h = g_ref[...] * cen * lax.rsqrt(var + 1e-5) + be_ref[...]
    h = jnp.maximum(h, 0.0)
    gids = lax.broadcasted_iota(jnp.int32, (1, G), 1)
    onehot = (batch_ref[...] == gids).astype(jnp.float32)
    counts = jnp.sum(onehot, axis=0)
    sums = lax.dot_general(onehot, h, (((0,), (0,)), ((), ())),
                           preferred_element_type=jnp.float32)
    pooled = sums / jnp.maximum(counts, 1.0)[:, None]
    logits = jnp.dot(pooled, wl_ref[...],
                     preferred_element_type=jnp.float32) + bl_ref[...]
    mx = jnp.max(logits, axis=1, keepdims=True)
    e = jnp.exp(logits - mx)
    out_ref[...] = e / jnp.sum(e, axis=1, keepdims=True)


def _stage_final(a0, a1, dinv, b, g, be, batch, wl, bl):
    hw = b.shape[0]
    return pl.pallas_call(
        _stage_final_body,
        out_shape=jax.ShapeDtypeStruct((G, C), jnp.float32),
    )(a0, a1, dinv, b.reshape(1, hw), g.reshape(1, hw), be.reshape(1, hw),
      batch.reshape(N, 1), wl, bl.reshape(1, C))


def kernel(x, edge_index, batch, W1, b1, g1, be1, W2, b2, g2, be2,
           W4, b4, g4, be4, W3, b3, g3, be3, Wl, bl):
    loop = jnp.arange(N, dtype=jnp.int32)
    pad = EPAD - (E + N)
    src = jnp.concatenate(
        [edge_index[0], loop, jnp.zeros((pad,), jnp.int32)])
    dst = jnp.concatenate(
        [edge_index[1], loop,
         N + (jnp.arange(pad, dtype=jnp.int32) % (NPAD - N))])
    src4 = src.reshape(NW, NSTEPS, STEP)
    dst4 = dst.reshape(NW, NSTEPS, STEP)
    dst3 = dst4

    zeros1 = jnp.zeros((NPAD,), jnp.float32)
    zeros128 = jnp.zeros((NPAD, 128), jnp.float32)

    degp = _deg(dst3, zeros1)
    dinv, zs = _stage0(degp[0, :N, None], degp[1, :N, None], x, W1)

    a = _agg128(zs, src4, dst4, zeros128)
    zs = _stage_mid(a[0], a[1], dinv, b1, g1, be1, W2)
    a = _agg128(zs, src4, dst4, zeros128)
    zs = _stage_mid(a[0], a[1], dinv, b2, g2, be2, W4)
    a = _agg128(zs, src4, dst4, zeros128)
    zs = _stage_mid(a[0], a[1], dinv, b4, g4, be4,
                    jnp.pad(W3, ((0, 0), (0, 64))))
    a = _agg128(zs, src4, dst4, zeros128)
    return _stage_final(a[0], a[1], dinv, b3, g3, be3, batch, Wl, bl)


# speedup vs baseline: 3.0340x; 1.0898x over previous
import functools

import jax
import jax.numpy as jnp
from jax import lax
from jax.experimental import pallas as pl
from jax.experimental.pallas import tpu as pltpu
from jax.experimental.pallas import tpu_sc as plsc

N = 10000
E = 320000
D = 128
G = 64
C = 10

NPAD = 10240
RPT = NPAD // 16
NW = 32
STEP = 128
NSTEPS = 81
NS_FAST = 95
NS_SLOW = 67
NS_MAX = 96
FAST_CORE = 1
EPAD = 16 * (NS_FAST + NS_SLOW) * STEP

_mesh = plsc.VectorSubcoreMesh(core_axis_name="c", subcore_axis_name="s")


@functools.partial(
    pl.kernel,
    mesh=_mesh,
    out_type=jax.ShapeDtypeStruct((2, NPAD), jnp.float32),
    scratch_types=[
        pltpu.VMEM((NSTEPS, STEP), jnp.int32),
        pltpu.VMEM((STEP,), jnp.float32),
        pltpu.VMEM_SHARED((NPAD,), jnp.float32),
        pltpu.SemaphoreType.DMA,
    ],
)
def _deg(dst_hbm, zeros1_hbm, out_hbm, dst_v, ones_v, acc, sem):
    c = lax.axis_index("c")
    s = lax.axis_index("s")
    wid = s * 2 + c
    for i in range(STEP // 16):
        ones_v[pl.ds(i * 16, 16)] = jnp.ones((16,), jnp.float32)
    pltpu.sync_copy(zeros1_hbm.at[pl.ds(s * RPT, RPT)],
                    acc.at[pl.ds(s * RPT, RPT)])
    pltpu.sync_copy(dst_hbm.at[wid], dst_v)
    plsc.subcore_barrier()

    def body(j, carry):
        pltpu.sync_copy(ones_v, acc.at[dst_v.at[j]], add=True)
        return carry

    lax.fori_loop(0, NSTEPS, body, 0)
    plsc.subcore_barrier()
    pltpu.sync_copy(acc.at[pl.ds(s * RPT, RPT)],
                    out_hbm.at[c, pl.ds(s * RPT, RPT)])


def _make_agg(H):
    @functools.partial(
        pl.kernel,
        mesh=_mesh,
        out_type=jax.ShapeDtypeStruct((2, NPAD, H), jnp.float32),
        scratch_types=[
            pltpu.VMEM((NS_MAX, STEP), jnp.int32),
            pltpu.VMEM((NS_MAX, STEP), jnp.int32),
            pltpu.VMEM((STEP, H), jnp.float32),
            pltpu.VMEM_SHARED((NPAD, H), jnp.float32),
            pltpu.SemaphoreType.DMA,
        ],
    )
    def _agg(zs_hbm, src_hbm, dst_hbm, zeros_hbm, out_hbm,
             src_v, dst_v, rows0, acc, gsem0):
        c = lax.axis_index("c")
        s = lax.axis_index("s")
        ns = jnp.where(c == FAST_CORE, NS_FAST, NS_SLOW)
        pltpu.sync_copy(zeros_hbm.at[pl.ds(s * RPT, RPT)],
                        acc.at[pl.ds(s * RPT, RPT)])
        pltpu.sync_copy(src_hbm.at[c, s], src_v)
        pltpu.sync_copy(dst_hbm.at[c, s], dst_v)
        plsc.subcore_barrier()

        def body(j, carry):
            pltpu.async_copy(zs_hbm.at[src_v.at[j]], rows0, gsem0).wait()
            pltpu.sync_copy(rows0, acc.at[dst_v.at[j]], add=True)
            return carry

        lax.fori_loop(0, ns, body, 0)
        plsc.subcore_barrier()
        pltpu.sync_copy(acc.at[pl.ds(s * RPT, RPT)],
                        out_hbm.at[c, pl.ds(s * RPT, RPT)])

    return _agg


_agg128 = _make_agg(128)


def _stage0_body(d0_ref, d1_ref, x_ref, w_ref, dinv_ref, zs_ref):
    deg = d0_ref[...] + d1_ref[...]
    dinv = lax.rsqrt(deg)
    dinv_ref[...] = dinv
    zs_ref[...] = jnp.dot(x_ref[...] * dinv, w_ref[...],
                          preferred_element_type=jnp.float32)


def _stage0(d0, d1, x, w):
    return pl.pallas_call(
        _stage0_body,
        out_shape=[jax.ShapeDtypeStruct((N, 1), jnp.float32),
                   jax.ShapeDtypeStruct((N, D), jnp.float32)],
    )(d0, d1, x, w)


def _stage_mid_body(a0_ref, a1_ref, dinv_ref, b_ref, g_ref, be_ref, w_ref,
                    zs_ref):
    dinv = dinv_ref[...]
    pre = (a0_ref[:N] + a1_ref[:N]) * dinv + b_ref[...]
    m = jnp.mean(pre, axis=0, keepdims=True)
    cen = pre - m
    var = jnp.mean(cen * cen, axis=0, keepdims=True)
    h = g_ref[...] * cen * lax.rsqrt(var + 1e-5) + be_ref[...]
    h = jnp.maximum(h, 0.0)
    zs_ref[...] = jnp.dot(h * dinv, w_ref[...],
                          preferred_element_type=jnp.float32)


def _stage_mid(a0, a1, dinv, b, g, be, w):
    hin, hout = w.shape
    return pl.pallas_call(
        _stage_mid_body,
        out_shape=jax.ShapeDtypeStruct((N, hout), jnp.float32),
    )(a0, a1, dinv, b.reshape(1, hin), g.reshape(1, hin), be.reshape(1, hin), w)


def _stage_final_body(a0_ref, a1_ref, dinv_ref, b_ref, g_ref, be_ref,
                      batch_ref, wl_ref, bl_ref, out_ref):
    pre = (a0_ref[:N, :64] + a1_ref[:N, :64]) * dinv_ref[...] + b_ref[...]
    m = jnp.mean(pre, axis=0, keepdims=True)
    cen = pre - m
    var = jnp.mean(cen * cen, axis=0, keepdims=True)
    h = g_ref[...] * cen * lax.rsqrt(var + 1e-5) + be_ref[...]
    h = jnp.maximum(h, 0.0)
    gids = lax.broadcasted_iota(jnp.int32, (1, G), 1)
    onehot = (batch_ref[...] == gids).astype(jnp.float32)
    counts = jnp.sum(onehot, axis=0)
    sums = lax.dot_general(onehot, h, (((0,), (0,)), ((), ())),
                           preferred_element_type=jnp.float32)
    pooled = sums / jnp.maximum(counts, 1.0)[:, None]
    logits = jnp.dot(pooled, wl_ref[...],
                     preferred_element_type=jnp.float32) + bl_ref[...]
    mx = jnp.max(logits, axis=1, keepdims=True)
    e = jnp.exp(logits - mx)
    out_ref[...] = e / jnp.sum(e, axis=1, keepdims=True)


def _stage_final(a0, a1, dinv, b, g, be, batch, wl, bl):
    hw = b.shape[0]
    return pl.pallas_call(
        _stage_final_body,
        out_shape=jax.ShapeDtypeStruct((G, C), jnp.float32),
    )(a0, a1, dinv, b.reshape(1, hw), g.reshape(1, hw), be.reshape(1, hw),
      batch.reshape(N, 1), wl, bl.reshape(1, C))


def kernel(x, edge_index, batch, W1, b1, g1, be1, W2, b2, g2, be2,
           W4, b4, g4, be4, W3, b3, g3, be3, Wl, bl):
    loop = jnp.arange(N, dtype=jnp.int32)
    pad = EPAD - (E + N)
    src = jnp.concatenate(
        [edge_index[0], loop, jnp.zeros((pad,), jnp.int32)])
    dst = jnp.concatenate(
        [edge_index[1], loop,
         N + (jnp.arange(pad, dtype=jnp.int32) % (NPAD - N))])
    nf = 16 * NS_FAST * STEP

    def _split(a, fill):
        f = a[:nf].reshape(16, NS_FAST, STEP)
        sl = a[nf:].reshape(16, NS_SLOW, STEP)
        f = jnp.pad(f, ((0, 0), (0, NS_MAX - NS_FAST), (0, 0)),
                    constant_values=fill)
        sl = jnp.pad(sl, ((0, 0), (0, NS_MAX - NS_SLOW), (0, 0)),
                     constant_values=fill)
        pair = [None, None]
        pair[FAST_CORE] = f
        pair[1 - FAST_CORE] = sl
        return jnp.stack(pair)

    src4 = _split(src, 0)
    dst4 = _split(dst, N)
    dst3 = dst.reshape(NW, NSTEPS, STEP)

    zeros1 = jnp.zeros((NPAD,), jnp.float32)
    zeros128 = jnp.zeros((NPAD, 128), jnp.float32)

    degp = _deg(dst3, zeros1)
    dinv, zs = _stage0(degp[0, :N, None], degp[1, :N, None], x, W1)

    a = _agg128(zs, src4, dst4, zeros128)
    zs = _stage_mid(a[0], a[1], dinv, b1, g1, be1, W2)
    a = _agg128(zs, src4, dst4, zeros128)
    zs = _stage_mid(a[0], a[1], dinv, b2, g2, be2, W4)
    a = _agg128(zs, src4, dst4, zeros128)
    zs = _stage_mid(a[0], a[1], dinv, b4, g4, be4,
                    jnp.pad(W3, ((0, 0), (0, 64))))
    a = _agg128(zs, src4, dst4, zeros128)
    return _stage_final(a[0], a[1], dinv, b3, g3, be3, batch, Wl, bl)
